# baseline (speedup 1.0000x reference)
import functools

import jax
import jax.numpy as jnp
from jax import lax
from jax.experimental import pallas as pl
from jax.experimental.pallas import tpu as pltpu
from jax.experimental.pallas import tpu_sc as plsc

ROWS = 4 * 4096
D = 2048
NC = 2
NS = 16
NW = NC * NS
CHUNK = ROWS // NW
R = 16
NG = CHUNK // R
_THRESH = 1258291


def _mask16(rows_i32):
    ks0 = jnp.int32(0)
    ks1 = jnp.int32(42)
    ks2 = jnp.int32(0x1BD11BDA ^ 42)
    ks = (ks0, ks1, ks2)
    rot_a = (13, 15, 26, 6)
    rot_b = (17, 29, 16, 24)

    x0 = jnp.zeros_like(rows_i32) + ks0
    x1 = rows_i32 + ks1
    for g in range(5):
        for r in (rot_a if g % 2 == 0 else rot_b):
            x0 = x0 + x1
            x1 = (x1 << r) | lax.shift_right_logical(x1, 32 - r)
            x1 = x1 ^ x0
        x0 = x0 + ks[(g + 1) % 3]
        x1 = x1 + ks[(g + 2) % 3] + jnp.int32(g + 1)
    bits = x0 ^ x1
    shifted = lax.shift_right_logical(bits, 9)
    return jnp.where(shifted <= jnp.int32(_THRESH), jnp.int32(1), jnp.int32(0))


def _sc_body(x_hbm, mv_hbm, out_hbm, buf, mask_v, mv_sh, sem):
    cid = lax.axis_index("c")
    sid = lax.axis_index("s")
    wid = sid * NC + cid
    base = wid * CHUNK

    @pl.when(sid == 0)
    def _():
        pltpu.sync_copy(mv_hbm, mv_sh)

    def mk(j, carry):
        rows = base + j * R + lax.broadcasted_iota(jnp.int32, (16,), 0)
        mask_v[pl.ds(pl.multiple_of(j * R, R), 16)] = _mask16(rows)
        return carry

    lax.fori_loop(0, NG, mk, 0)
    plsc.subcore_barrier()

    def grp(g, carry):
        rowbase = base + g * R
        pltpu.sync_copy(x_hbm.at[pl.ds(rowbase, R)], buf)
        m16 = mask_v[pl.ds(pl.multiple_of(g * R, R), 16)]
        for k in range(R):
            @pl.when(m16[k] != 0)
            def _():
                pltpu.sync_copy(mv_sh, buf.at[k])

        pltpu.sync_copy(buf, out_hbm.at[pl.ds(rowbase, R)])
        return carry

    lax.fori_loop(0, NG, grp, 0)


@jax.jit
def kernel(inputs, mask_value):
    x = inputs.reshape(ROWS, D)
    mesh = plsc.VectorSubcoreMesh(core_axis_name="c", subcore_axis_name="s")
    out = pl.kernel(
        _sc_body,
        out_type=jax.ShapeDtypeStruct((ROWS, D), jnp.float32),
        mesh=mesh,
        scratch_types=[
            pltpu.VMEM((R, D), jnp.float32),
            pltpu.VMEM((CHUNK,), jnp.int32),
            pltpu.VMEM_SHARED((D,), jnp.float32),
            pltpu.SemaphoreType.DMA,
        ],
    )(x, mask_value)
    return out.reshape(inputs.shape)


# speedup vs baseline: 1.3888x; 1.3888x over previous
import functools

import jax
import jax.numpy as jnp
from jax import lax
from jax.experimental import pallas as pl
from jax.experimental.pallas import tpu as pltpu
from jax.experimental.pallas import tpu_sc as plsc

ROWS = 4 * 4096
D = 2048
NC = 2
NS = 16
NW = NC * NS
CHUNK = ROWS // NW
R = 8
NG = CHUNK // R
NBUF = 4
P = 2
_THRESH = 1258291


def _mask16(rows_i32):
    ks0 = jnp.int32(0)
    ks1 = jnp.int32(42)
    ks2 = jnp.int32(0x1BD11BDA ^ 42)
    ks = (ks0, ks1, ks2)
    rot_a = (13, 15, 26, 6)
    rot_b = (17, 29, 16, 24)

    x0 = jnp.zeros_like(rows_i32) + ks0
    x1 = rows_i32 + ks1
    for g in range(5):
        for r in (rot_a if g % 2 == 0 else rot_b):
            x0 = x0 + x1
            x1 = (x1 << r) | lax.shift_right_logical(x1, 32 - r)
            x1 = x1 ^ x0
        x0 = x0 + ks[(g + 1) % 3]
        x1 = x1 + ks[(g + 2) % 3] + jnp.int32(g + 1)
    bits = x0 ^ x1
    shifted = lax.shift_right_logical(bits, 9)
    return jnp.where(shifted <= jnp.int32(_THRESH), jnp.int32(1), jnp.int32(0))


def _sc_body(x_hbm, mv_hbm, out_hbm, buf, mask_v, mv_sh, *sems):
    semg = sems[:NBUF]
    sems_ = sems[NBUF:]
    cid = lax.axis_index("c")
    sid = lax.axis_index("s")
    wid = sid * NC + cid
    base = wid * CHUNK

    @pl.when(sid == 0)
    def _():
        pltpu.sync_copy(mv_hbm, mv_sh)

    def mk(j, carry):
        rows = base + j * 16 + lax.broadcasted_iota(jnp.int32, (16,), 0)
        mask_v[pl.ds(pl.multiple_of(j * 16, 16), 16)] = _mask16(rows)
        return carry

    lax.fori_loop(0, CHUNK // 16, mk, 0)
    plsc.subcore_barrier()

    def step(g, carry):
        slot = jnp.bitwise_and(g, NBUF - 1)

        @pl.when(g < NG)
        def _gather():
            rowbase = base + g * R
            for j in range(NBUF):
                @pl.when(slot == j)
                def _(j=j):
                    @pl.when(g >= NBUF)
                    def _():
                        pltpu.make_async_copy(
                            buf.at[pl.ds(j * R, R)],
                            out_hbm.at[pl.ds(rowbase, R)],
                            sems_[j],
                        ).wait()

                    pltpu.async_copy(
                        x_hbm.at[pl.ds(rowbase, R)],
                        buf.at[pl.ds(j * R, R)],
                        semg[j],
                    )

        @pl.when(g >= P)
        def _process():
            gp = g - P
            slotp = jnp.bitwise_and(gp, NBUF - 1)
            rowbase = base + gp * R
            for j in range(NBUF):
                @pl.when(slotp == j)
                def _(j=j):
                    pltpu.make_async_copy(
                        x_hbm.at[pl.ds(rowbase, R)],
                        buf.at[pl.ds(j * R, R)],
                        semg[j],
                    ).wait()

            m16 = mask_v[pl.ds(gp * R, 16)]
            for k in range(R):
                @pl.when(m16[k] != 0)
                def _(k=k):
                    pltpu.sync_copy(mv_sh, buf.at[slotp * R + k])

            for j in range(NBUF):
                @pl.when(slotp == j)
                def _(j=j):
                    pltpu.async_copy(
                        buf.at[pl.ds(j * R, R)],
                        out_hbm.at[pl.ds(rowbase, R)],
                        sems_[j],
                    )

        return carry

    lax.fori_loop(0, NG + P, step, 0)

    for j in range(NBUF):
        pltpu.make_async_copy(
            buf.at[pl.ds(j * R, R)],
            out_hbm.at[pl.ds(base, R)],
            sems_[j],
        ).wait()


@jax.jit
def kernel(inputs, mask_value):
    x = inputs.reshape(ROWS, D)
    mesh = plsc.VectorSubcoreMesh(core_axis_name="c", subcore_axis_name="s")
    out = pl.kernel(
        _sc_body,
        out_type=jax.ShapeDtypeStruct((ROWS, D), jnp.float32),
        mesh=mesh,
        scratch_types=(
            [
                pltpu.VMEM((NBUF * R, D), jnp.float32),
                pltpu.VMEM((CHUNK + 16, ), jnp.int32),
                pltpu.VMEM_SHARED((D,), jnp.float32),
            ]
            + [pltpu.SemaphoreType.DMA] * (2 * NBUF)
        ),
    )(x, mask_value)
    return out.reshape(inputs.shape)
